# Initial kernel scaffold; baseline (speedup 1.0000x reference)
import jax
import jax.numpy as jnp
from jax.experimental import pallas as pl


def kernel(float_fields, token_fields, token_seq_field, float_emb_table, token_emb_table, token_seq_emb_table, bias, offsets):
    raise NotImplementedError("write your pallas kernel here")


# speedup vs baseline: 47.3506x; 47.3506x over previous
import functools

import jax
import jax.numpy as jnp
from jax import lax
from jax.experimental import pallas as pl
from jax.experimental.pallas import tpu as pltpu
from jax.experimental.pallas import tpu_sc as plsc

B = 16384
NF = 13
NT = 26
VT = 100000
VS = 100000
LS = 50

_info = plsc.get_sparse_core_info()
NC = _info.num_cores
NS = _info.num_subcores
LANES = _info.num_lanes
NW = NC * NS
CH = B // NW
NJ = CH // LANES

_mesh = plsc.VectorSubcoreMesh(core_axis_name="c", subcore_axis_name="s")


@functools.partial(
    pl.kernel,
    mesh=_mesh,
    out_type=jax.ShapeDtypeStruct((B,), jnp.float32),
    scratch_types=[
        pltpu.VMEM((NT * CH,), jnp.int32),
        pltpu.VMEM((NT * CH,), jnp.float32),
        pltpu.VMEM((LS * CH,), jnp.int32),
        pltpu.VMEM((LS * CH,), jnp.float32),
        pltpu.VMEM((NF * CH,), jnp.float32),
        pltpu.VMEM((CH,), jnp.float32),
        pltpu.VMEM((NT * LANES,), jnp.int32),
        pltpu.VMEM((NF * LANES,), jnp.float32),
        pltpu.VMEM((LANES,), jnp.float32),
        pltpu.SemaphoreType.DMA,
        pltpu.SemaphoreType.DMA,
    ],
)
def _fm_sc(tf_hbm, sf_hbm, ff_hbm, tok_tab, seq_tab, fw_hbm, bias_hbm,
           off_hbm, out_hbm,
           tok_idx, tok_val, seq_idx, seq_val, ff_v, out_v,
           off_v, fw_v, bias_v, sem_t, sem_q):
    wid = lax.axis_index("s") * NC + lax.axis_index("c")
    base = wid * CH

    pltpu.sync_copy(off_hbm, off_v)
    pltpu.sync_copy(fw_hbm, fw_v)
    pltpu.sync_copy(bias_hbm, bias_v)
    pltpu.sync_copy(tf_hbm.at[wid], tok_idx)
    pltpu.sync_copy(sf_hbm.at[wid], seq_idx)
    pltpu.sync_copy(ff_hbm.at[wid], ff_v)

    def off_body(i, carry):
        sl = pl.ds(i * LANES, LANES)
        tok_idx[sl] = tok_idx[sl] + off_v[pl.ds((i // NJ) * LANES, LANES)]
        return carry

    lax.fori_loop(0, NT * NJ, off_body, 0)

    cp_t = pltpu.async_copy(tok_tab.at[tok_idx], tok_val, sem_t)
    cp_q = pltpu.async_copy(seq_tab.at[seq_idx], seq_val, sem_q)
    cp_t.wait()
    cp_q.wait()

    def compute(j, carry):
        b16 = pl.ds(j * LANES, LANES)
        acc = bias_v[pl.ds(0, LANES)]
        for t in range(NT):
            acc = acc + tok_val[pl.ds(t * CH + j * LANES, LANES)]
        for l in range(LS):
            sl = pl.ds(l * CH + j * LANES, LANES)
            sv = seq_idx[sl]
            acc = acc + jnp.where(sv != 0, seq_val[sl], 0.0)
        for f in range(NF):
            acc = acc + ff_v[pl.ds(f * CH + j * LANES, LANES)] * fw_v[pl.ds(f * LANES, LANES)]
        out_v[b16] = acc
        return carry

    lax.fori_loop(0, NJ, compute, 0)
    pltpu.sync_copy(out_v, out_hbm.at[pl.ds(base, CH)])


def _field_major(x, nfields):
    return x.T.reshape(nfields, NW, CH).transpose(1, 0, 2).reshape(NW, nfields * CH)


def kernel(float_fields, token_fields, token_seq_field, float_emb_table,
           token_emb_table, token_seq_emb_table, bias, offsets):
    tf_w = _field_major(token_fields, NT)
    sf_w = _field_major(token_seq_field, LS)
    ff_w = _field_major(float_fields, NF)
    fw_rep = jnp.repeat(float_emb_table.reshape(-1), LANES)
    bias_rep = jnp.broadcast_to(bias.reshape(1), (LANES,))
    off_rep = jnp.repeat(offsets, LANES)
    out = _fm_sc(tf_w, sf_w, ff_w,
                 token_emb_table.reshape(-1),
                 token_seq_emb_table.reshape(-1),
                 fw_rep, bias_rep, off_rep)
    return out.reshape(B, 1)


# speedup vs baseline: 58.5921x; 1.2374x over previous
import functools

import jax
import jax.numpy as jnp
from jax import lax
from jax.experimental import pallas as pl
from jax.experimental.pallas import tpu as pltpu
from jax.experimental.pallas import tpu_sc as plsc

B = 16384
NF = 13
NT = 26
VT = 100000
VS = 100000
LS = 50

_info = plsc.get_sparse_core_info()
NC = _info.num_cores
NS = _info.num_subcores
LANES = _info.num_lanes
NW = NC * NS
CH = B // NW
NJ = CH // LANES

_mesh = plsc.VectorSubcoreMesh(core_axis_name="c", subcore_axis_name="s")


@functools.partial(
    pl.kernel,
    mesh=_mesh,
    out_type=jax.ShapeDtypeStruct((B,), jnp.float32),
    scratch_types=[
        pltpu.VMEM((LS * CH,), jnp.int32),
        pltpu.VMEM((LS * CH,), jnp.float32),
        pltpu.VMEM((NF * CH,), jnp.float32),
        pltpu.VMEM((CH,), jnp.float32),
        pltpu.VMEM((NF * LANES,), jnp.float32),
        pltpu.VMEM((LANES,), jnp.float32),
        pltpu.SemaphoreType.DMA,
    ],
)
def _fm_seq(sf_hbm, ff_hbm, seq_tab, fw_hbm, bias_hbm, part_hbm,
            seq_idx, seq_val, ff_v, part_v, fw_v, bias_v, sem_q):
    wid = lax.axis_index("s") * NC + lax.axis_index("c")
    base = wid * CH

    pltpu.sync_copy(sf_hbm.at[wid], seq_idx)
    cp_q = pltpu.async_copy(seq_tab.at[seq_idx], seq_val, sem_q)
    pltpu.sync_copy(ff_hbm.at[wid], ff_v)
    pltpu.sync_copy(fw_hbm, fw_v)
    pltpu.sync_copy(bias_hbm, bias_v)
    cp_q.wait()

    def compute(j, carry):
        acc = bias_v[pl.ds(0, LANES)]
        for l in range(LS):
            sl = pl.ds(l * CH + j * LANES, LANES)
            sv = seq_idx[sl]
            acc = acc + jnp.where(sv != 0, seq_val[sl], 0.0)
        for f in range(NF):
            acc = acc + ff_v[pl.ds(f * CH + j * LANES, LANES)] * fw_v[pl.ds(f * LANES, LANES)]
        part_v[pl.ds(j * LANES, LANES)] = acc
        return carry

    lax.fori_loop(0, NJ, compute, 0)
    pltpu.sync_copy(part_v, part_hbm.at[pl.ds(base, CH)])


@functools.partial(
    pl.kernel,
    mesh=_mesh,
    out_type=jax.ShapeDtypeStruct((B,), jnp.float32),
    scratch_types=[
        pltpu.VMEM((NT * CH,), jnp.int32),
        pltpu.VMEM((NT * CH,), jnp.float32),
        pltpu.VMEM((CH,), jnp.float32),
        pltpu.VMEM((CH,), jnp.float32),
        pltpu.VMEM((NT * LANES,), jnp.int32),
        pltpu.SemaphoreType.DMA,
    ],
)
def _fm_tok(tf_hbm, tok_tab, off_hbm, part_hbm, out_hbm,
            tok_idx, tok_val, part_v, out_v, off_v, sem_t):
    wid = lax.axis_index("s") * NC + lax.axis_index("c")
    base = wid * CH

    pltpu.sync_copy(tf_hbm.at[wid], tok_idx)
    pltpu.sync_copy(off_hbm, off_v)

    def off_body(i, carry):
        sl = pl.ds(i * LANES, LANES)
        tok_idx[sl] = tok_idx[sl] + off_v[pl.ds((i // NJ) * LANES, LANES)]
        return carry

    lax.fori_loop(0, NT * NJ, off_body, 0)

    cp_t = pltpu.async_copy(tok_tab.at[tok_idx], tok_val, sem_t)
    pltpu.sync_copy(part_hbm.at[pl.ds(base, CH)], part_v)
    cp_t.wait()

    def compute(j, carry):
        acc = part_v[pl.ds(j * LANES, LANES)]
        for t in range(NT):
            acc = acc + tok_val[pl.ds(t * CH + j * LANES, LANES)]
        out_v[pl.ds(j * LANES, LANES)] = acc
        return carry

    lax.fori_loop(0, NJ, compute, 0)
    pltpu.sync_copy(out_v, out_hbm.at[pl.ds(base, CH)])


def _field_major(x, nfields):
    return x.T.reshape(nfields, NW, CH).transpose(1, 0, 2).reshape(NW, nfields * CH)


def kernel(float_fields, token_fields, token_seq_field, float_emb_table,
           token_emb_table, token_seq_emb_table, bias, offsets):
    tf_w = _field_major(token_fields, NT)
    sf_w = _field_major(token_seq_field, LS)
    ff_w = _field_major(float_fields, NF)
    fw_rep = jnp.repeat(float_emb_table.reshape(-1), LANES)
    bias_rep = jnp.broadcast_to(bias.reshape(1), (LANES,))
    off_rep = jnp.repeat(offsets, LANES)
    part = _fm_seq(sf_w, ff_w, token_seq_emb_table.reshape(-1),
                   fw_rep, bias_rep)
    out = _fm_tok(tf_w, token_emb_table.reshape(-1), off_rep, part)
    return out.reshape(B, 1)


# speedup vs baseline: 106.6332x; 1.8199x over previous
import functools

import jax
import jax.numpy as jnp
from jax import lax
from jax.experimental import pallas as pl
from jax.experimental.pallas import tpu as pltpu
from jax.experimental.pallas import tpu_sc as plsc

B = 16384
NF = 13
NT = 26
VT = 100000
VS = 100000
LS = 50

_info = plsc.get_sparse_core_info()
NC = _info.num_cores
NS = _info.num_subcores
LANES = _info.num_lanes
NW = NC * NS
CH = B // NW
NJ = CH // LANES

_mesh = plsc.VectorSubcoreMesh(core_axis_name="c", subcore_axis_name="s")


@functools.partial(
    pl.kernel,
    mesh=_mesh,
    out_type=jax.ShapeDtypeStruct((B,), jnp.float32),
    scratch_types=[
        pltpu.VMEM((LS * CH,), jnp.int32),
        pltpu.VMEM((LS * CH,), jnp.float32),
        pltpu.VMEM((NF * CH,), jnp.float32),
        pltpu.VMEM((CH,), jnp.float32),
        pltpu.VMEM((NF * LANES,), jnp.float32),
        pltpu.VMEM((LANES,), jnp.float32),
        pltpu.SemaphoreType.DMA,
    ],
)
def _fm_seq(sf_hbm, ff_hbm, seq_tab, fw_hbm, bias_hbm, part_hbm,
            seq_idx, seq_val, ff_v, part_v, fw_v, bias_v, sem_q):
    wid = lax.axis_index("s") * NC + lax.axis_index("c")
    base = wid * CH

    pltpu.sync_copy(sf_hbm.at[wid], seq_idx)
    cp_q = pltpu.async_copy(seq_tab.at[seq_idx], seq_val, sem_q)
    pltpu.sync_copy(ff_hbm.at[wid], ff_v)
    pltpu.sync_copy(fw_hbm, fw_v)
    pltpu.sync_copy(bias_hbm, bias_v)
    cp_q.wait()

    def compute(j, carry):
        acc = bias_v[pl.ds(0, LANES)]
        for l in range(LS):
            sl = pl.ds(l * CH + j * LANES, LANES)
            sv = seq_idx[sl]
            acc = acc + jnp.where(sv != 0, seq_val[sl], 0.0)
        for f in range(NF):
            acc = acc + ff_v[pl.ds(f * CH + j * LANES, LANES)] * fw_v[pl.ds(f * LANES, LANES)]
        part_v[pl.ds(j * LANES, LANES)] = acc
        return carry

    lax.fori_loop(0, NJ, compute, 0)
    pltpu.sync_copy(part_v, part_hbm.at[pl.ds(base, CH)])


@functools.partial(
    pl.kernel,
    mesh=_mesh,
    out_type=jax.ShapeDtypeStruct((B,), jnp.float32),
    scratch_types=[
        pltpu.VMEM((NT * CH,), jnp.int32),
        pltpu.VMEM((NT * CH,), jnp.float32),
        pltpu.VMEM((CH,), jnp.float32),
        pltpu.VMEM((CH,), jnp.float32),
        pltpu.SemaphoreType.DMA,
    ],
)
def _fm_tok(tf_hbm, *rest):
    tabs = rest[:NT]
    part_hbm, out_hbm, tok_idx, tok_val, part_v, out_v, sem_t = rest[NT:]
    wid = lax.axis_index("s") * NC + lax.axis_index("c")
    base = wid * CH

    pltpu.sync_copy(tf_hbm.at[wid], tok_idx)

    cps = []
    for t in range(NT):
        cps.append(pltpu.async_copy(
            tabs[t].at[tok_idx.at[pl.ds(t * CH, CH)]],
            tok_val.at[pl.ds(t * CH, CH)], sem_t))
    pltpu.sync_copy(part_hbm.at[pl.ds(base, CH)], part_v)
    for cp in cps:
        cp.wait()

    def compute(j, carry):
        acc = part_v[pl.ds(j * LANES, LANES)]
        for t in range(NT):
            acc = acc + tok_val[pl.ds(t * CH + j * LANES, LANES)]
        out_v[pl.ds(j * LANES, LANES)] = acc
        return carry

    lax.fori_loop(0, NJ, compute, 0)
    pltpu.sync_copy(out_v, out_hbm.at[pl.ds(base, CH)])


def _field_major(x, nfields):
    return x.T.reshape(nfields, NW, CH).transpose(1, 0, 2).reshape(NW, nfields * CH)


def kernel(float_fields, token_fields, token_seq_field, float_emb_table,
           token_emb_table, token_seq_emb_table, bias, offsets):
    tf_w = _field_major(token_fields, NT)
    sf_w = _field_major(token_seq_field, LS)
    ff_w = _field_major(float_fields, NF)
    fw_rep = jnp.repeat(float_emb_table.reshape(-1), LANES)
    bias_rep = jnp.broadcast_to(bias.reshape(1), (LANES,))
    del offsets
    part = _fm_seq(sf_w, ff_w, token_seq_emb_table.reshape(-1),
                   fw_rep, bias_rep)
    tabs = [token_emb_table[i * VT:(i + 1) * VT].reshape(-1)
            for i in range(NT)]
    out = _fm_tok(tf_w, *tabs, part)
    return out.reshape(B, 1)


# speedup vs baseline: 107.6450x; 1.0095x over previous
import functools

import jax
import jax.numpy as jnp
from jax import lax
from jax.experimental import pallas as pl
from jax.experimental.pallas import tpu as pltpu
from jax.experimental.pallas import tpu_sc as plsc

B = 16384
NF = 13
NT = 26
VT = 100000
VS = 100000
LS = 50

_info = plsc.get_sparse_core_info()
NC = _info.num_cores
NS = _info.num_subcores
LANES = _info.num_lanes
NW = NC * NS
CH = B // NW
NJ = CH // LANES

_mesh = plsc.VectorSubcoreMesh(core_axis_name="c", subcore_axis_name="s")

_WAVES_T = [range(0, 7), range(7, 14), range(14, 20), range(20, 26)]
_WAVES_S = [range(10 * w, 10 * (w + 1)) for w in range(5)]


@functools.partial(
    pl.kernel,
    mesh=_mesh,
    out_type=jax.ShapeDtypeStruct((B,), jnp.float32),
    scratch_types=[
        pltpu.VMEM((LS * CH,), jnp.int32),
        pltpu.VMEM((LS * CH,), jnp.float32),
        pltpu.VMEM((CH,), jnp.float32),
    ] + [pltpu.SemaphoreType.DMA] * len(_WAVES_S),
)
def _fm_seq(sf_hbm, seq_tab, part_hbm, seq_idx, seq_val, part_v, *sems):
    wid = lax.axis_index("s") * NC + lax.axis_index("c")
    base = wid * CH

    pltpu.sync_copy(sf_hbm.at[wid], seq_idx)
    waves = []
    for w, fields in enumerate(_WAVES_S):
        waves.append([
            pltpu.async_copy(seq_tab.at[seq_idx.at[pl.ds(l * CH, CH)]],
                             seq_val.at[pl.ds(l * CH, CH)], sems[w])
            for l in fields])

    for w, fields in enumerate(_WAVES_S):
        for cp in waves[w]:
            cp.wait()

        def acc_body(j, carry, fields=fields, first=(w == 0)):
            js = pl.ds(j * LANES, LANES)
            acc = jnp.zeros((LANES,), jnp.float32) if first else part_v[js]
            for l in fields:
                sl = pl.ds(l * CH + j * LANES, LANES)
                acc = acc + jnp.where(seq_idx[sl] != 0, seq_val[sl], 0.0)
            part_v[js] = acc
            return carry

        lax.fori_loop(0, NJ, acc_body, 0)

    pltpu.sync_copy(part_v, part_hbm.at[pl.ds(base, CH)])


@functools.partial(
    pl.kernel,
    mesh=_mesh,
    out_type=jax.ShapeDtypeStruct((B,), jnp.float32),
    scratch_types=[
        pltpu.VMEM((NT * CH,), jnp.int32),
        pltpu.VMEM((NT * CH,), jnp.float32),
        pltpu.VMEM((NF * CH,), jnp.float32),
        pltpu.VMEM((CH,), jnp.float32),
        pltpu.VMEM((CH,), jnp.float32),
        pltpu.VMEM((NF * LANES,), jnp.float32),
        pltpu.VMEM((LANES,), jnp.float32),
    ] + [pltpu.SemaphoreType.DMA] * len(_WAVES_T),
)
def _fm_tok(tf_hbm, *rest):
    tabs = rest[:NT]
    (ff_hbm, fw_hbm, bias_hbm, part_hbm, out_hbm,
     tok_idx, tok_val, ff_v, part_v, out_v, fw_v, bias_v) = rest[NT:NT + 12]
    sems = rest[NT + 12:]
    wid = lax.axis_index("s") * NC + lax.axis_index("c")
    base = wid * CH

    pltpu.sync_copy(tf_hbm.at[wid], tok_idx)
    waves = []
    for w, fields in enumerate(_WAVES_T):
        waves.append([
            pltpu.async_copy(tabs[t].at[tok_idx.at[pl.ds(t * CH, CH)]],
                             tok_val.at[pl.ds(t * CH, CH)], sems[w])
            for t in fields])

    pltpu.sync_copy(ff_hbm.at[wid], ff_v)
    pltpu.sync_copy(fw_hbm, fw_v)
    pltpu.sync_copy(bias_hbm, bias_v)
    pltpu.sync_copy(part_hbm.at[pl.ds(base, CH)], part_v)

    def base_body(j, carry):
        js = pl.ds(j * LANES, LANES)
        acc = part_v[js] + bias_v[pl.ds(0, LANES)]
        for f in range(NF):
            acc = acc + ff_v[pl.ds(f * CH + j * LANES, LANES)] * fw_v[pl.ds(f * LANES, LANES)]
        out_v[js] = acc
        return carry

    lax.fori_loop(0, NJ, base_body, 0)

    for w, fields in enumerate(_WAVES_T):
        for cp in waves[w]:
            cp.wait()

        def acc_body(j, carry, fields=fields):
            js = pl.ds(j * LANES, LANES)
            acc = out_v[js]
            for t in fields:
                acc = acc + tok_val[pl.ds(t * CH + j * LANES, LANES)]
            out_v[js] = acc
            return carry

        lax.fori_loop(0, NJ, acc_body, 0)

    pltpu.sync_copy(out_v, out_hbm.at[pl.ds(base, CH)])


def _field_major(x, nfields):
    return x.T.reshape(nfields, NW, CH).transpose(1, 0, 2).reshape(NW, nfields * CH)


def kernel(float_fields, token_fields, token_seq_field, float_emb_table,
           token_emb_table, token_seq_emb_table, bias, offsets):
    tf_w = _field_major(token_fields, NT)
    sf_w = _field_major(token_seq_field, LS)
    ff_w = _field_major(float_fields, NF)
    fw_rep = jnp.repeat(float_emb_table.reshape(-1), LANES)
    bias_rep = jnp.broadcast_to(bias.reshape(1), (LANES,))
    del offsets
    part = _fm_seq(sf_w, token_seq_emb_table.reshape(-1))
    tabs = [token_emb_table[i * VT:(i + 1) * VT].reshape(-1)
            for i in range(NT)]
    out = _fm_tok(tf_w, *tabs, ff_w, fw_rep, bias_rep, part)
    return out.reshape(B, 1)


# speedup vs baseline: 112.9878x; 1.0496x over previous
import functools

import jax
import jax.numpy as jnp
from jax import lax
from jax.experimental import pallas as pl
from jax.experimental.pallas import tpu as pltpu
from jax.experimental.pallas import tpu_sc as plsc

B = 16384
NF = 13
NT = 26
VT = 100000
VS = 100000
LS = 50

_info = plsc.get_sparse_core_info()
NC = _info.num_cores
NS = _info.num_subcores
LANES = _info.num_lanes
NW = NC * NS
CH = B // NW
NJ = CH // LANES

_mesh = plsc.VectorSubcoreMesh(core_axis_name="c", subcore_axis_name="s")

_WAVES_T = [range(0, 7), range(7, 14), range(14, 20), range(20, 26)]
_WAVES_S = [range(10 * w, 10 * (w + 1)) for w in range(5)]


@functools.partial(
    pl.kernel,
    mesh=_mesh,
    out_type=jax.ShapeDtypeStruct((B,), jnp.float32),
    scratch_types=[
        pltpu.VMEM((LS * CH,), jnp.int32),
        pltpu.VMEM((LS * CH,), jnp.float32),
        pltpu.VMEM((CH,), jnp.float32),
        pltpu.VMEM_SHARED((VS,), jnp.float32),
    ] + [pltpu.SemaphoreType.DMA] * len(_WAVES_S),
)
def _fm_seq(sf_hbm, seq_tab, part_hbm, seq_idx, seq_val, part_v, tab_sh,
            *sems):
    wid = lax.axis_index("s") * NC + lax.axis_index("c")
    base = wid * CH

    @pl.when(lax.axis_index("s") == 0)
    def _():
        pltpu.sync_copy(seq_tab, tab_sh)

    pltpu.sync_copy(sf_hbm.at[wid], seq_idx)
    plsc.subcore_barrier()
    waves = []
    for w, fields in enumerate(_WAVES_S):
        waves.append([
            pltpu.async_copy(tab_sh.at[seq_idx.at[pl.ds(l * CH, CH)]],
                             seq_val.at[pl.ds(l * CH, CH)], sems[w])
            for l in fields])

    for w, fields in enumerate(_WAVES_S):
        for cp in waves[w]:
            cp.wait()

        def acc_body(j, carry, fields=fields, first=(w == 0)):
            js = pl.ds(j * LANES, LANES)
            acc = jnp.zeros((LANES,), jnp.float32) if first else part_v[js]
            for l in fields:
                sl = pl.ds(l * CH + j * LANES, LANES)
                acc = acc + jnp.where(seq_idx[sl] != 0, seq_val[sl], 0.0)
            part_v[js] = acc
            return carry

        lax.fori_loop(0, NJ, acc_body, 0)

    pltpu.sync_copy(part_v, part_hbm.at[pl.ds(base, CH)])


@functools.partial(
    pl.kernel,
    mesh=_mesh,
    out_type=jax.ShapeDtypeStruct((B,), jnp.float32),
    scratch_types=[
        pltpu.VMEM((NT * CH,), jnp.int32),
        pltpu.VMEM((NT * CH,), jnp.float32),
        pltpu.VMEM((NF * CH,), jnp.float32),
        pltpu.VMEM((CH,), jnp.float32),
        pltpu.VMEM((CH,), jnp.float32),
        pltpu.VMEM((NF * LANES,), jnp.float32),
        pltpu.VMEM((LANES,), jnp.float32),
    ] + [pltpu.SemaphoreType.DMA] * len(_WAVES_T),
)
def _fm_tok(tf_hbm, *rest):
    tabs = rest[:NT]
    (ff_hbm, fw_hbm, bias_hbm, part_hbm, out_hbm,
     tok_idx, tok_val, ff_v, part_v, out_v, fw_v, bias_v) = rest[NT:NT + 12]
    sems = rest[NT + 12:]
    wid = lax.axis_index("s") * NC + lax.axis_index("c")
    base = wid * CH

    pltpu.sync_copy(tf_hbm.at[wid], tok_idx)
    waves = []
    for w, fields in enumerate(_WAVES_T):
        waves.append([
            pltpu.async_copy(tabs[t].at[tok_idx.at[pl.ds(t * CH, CH)]],
                             tok_val.at[pl.ds(t * CH, CH)], sems[w])
            for t in fields])

    pltpu.sync_copy(ff_hbm.at[wid], ff_v)
    pltpu.sync_copy(fw_hbm, fw_v)
    pltpu.sync_copy(bias_hbm, bias_v)
    pltpu.sync_copy(part_hbm.at[pl.ds(base, CH)], part_v)

    def base_body(j, carry):
        js = pl.ds(j * LANES, LANES)
        acc = part_v[js] + bias_v[pl.ds(0, LANES)]
        for f in range(NF):
            acc = acc + ff_v[pl.ds(f * CH + j * LANES, LANES)] * fw_v[pl.ds(f * LANES, LANES)]
        out_v[js] = acc
        return carry

    lax.fori_loop(0, NJ, base_body, 0)

    for w, fields in enumerate(_WAVES_T):
        for cp in waves[w]:
            cp.wait()

        def acc_body(j, carry, fields=fields):
            js = pl.ds(j * LANES, LANES)
            acc = out_v[js]
            for t in fields:
                acc = acc + tok_val[pl.ds(t * CH + j * LANES, LANES)]
            out_v[js] = acc
            return carry

        lax.fori_loop(0, NJ, acc_body, 0)

    pltpu.sync_copy(out_v, out_hbm.at[pl.ds(base, CH)])


def _field_major(x, nfields):
    return x.T.reshape(nfields, NW, CH).transpose(1, 0, 2).reshape(NW, nfields * CH)


def kernel(float_fields, token_fields, token_seq_field, float_emb_table,
           token_emb_table, token_seq_emb_table, bias, offsets):
    tf_w = _field_major(token_fields, NT)
    sf_w = _field_major(token_seq_field, LS)
    ff_w = _field_major(float_fields, NF)
    fw_rep = jnp.repeat(float_emb_table.reshape(-1), LANES)
    bias_rep = jnp.broadcast_to(bias.reshape(1), (LANES,))
    del offsets
    part = _fm_seq(sf_w, token_seq_emb_table.reshape(-1))
    tabs = [token_emb_table[i * VT:(i + 1) * VT].reshape(-1)
            for i in range(NT)]
    out = _fm_tok(tf_w, *tabs, ff_w, fw_rep, bias_rep, part)
    return out.reshape(B, 1)


# speedup vs baseline: 121.2814x; 1.0734x over previous
import functools

import jax
import jax.numpy as jnp
from jax import lax
from jax.experimental import pallas as pl
from jax.experimental.pallas import tpu as pltpu
from jax.experimental.pallas import tpu_sc as plsc

B = 16384
NF = 13
NT = 26
VT = 100000
VS = 100000
LS = 50

_info = plsc.get_sparse_core_info()
NC = _info.num_cores
NS = _info.num_subcores
LANES = _info.num_lanes
NW = NC * NS
CH = B // NW
NJ = CH // LANES

_mesh = plsc.VectorSubcoreMesh(core_axis_name="c", subcore_axis_name="s")

_NTH = 13
_WAVES_T = [range(0, 7), range(7, 13)]
_WAVES_S = [range(10 * w, 10 * (w + 1)) for w in range(5)]


@functools.partial(
    pl.kernel,
    mesh=_mesh,
    out_type=jax.ShapeDtypeStruct((B,), jnp.float32),
    scratch_types=[
        pltpu.VMEM((LS * CH,), jnp.int32),
        pltpu.VMEM((LS * CH,), jnp.float32),
        pltpu.VMEM((CH,), jnp.float32),
        pltpu.VMEM_SHARED((VS,), jnp.float32),
    ] + [pltpu.SemaphoreType.DMA] * len(_WAVES_S),
)
def _fm_seq(sf_hbm, seq_tab, part_hbm, seq_idx, seq_val, part_v, tab_sh,
            *sems):
    wid = lax.axis_index("s") * NC + lax.axis_index("c")
    base = wid * CH

    @pl.when(lax.axis_index("s") == 0)
    def _():
        pltpu.sync_copy(seq_tab, tab_sh)

    pltpu.sync_copy(sf_hbm.at[wid], seq_idx)
    plsc.subcore_barrier()
    waves = []
    for w, fields in enumerate(_WAVES_S):
        waves.append([
            pltpu.async_copy(tab_sh.at[seq_idx.at[pl.ds(l * CH, CH)]],
                             seq_val.at[pl.ds(l * CH, CH)], sems[w])
            for l in fields])

    for w, fields in enumerate(_WAVES_S):
        for cp in waves[w]:
            cp.wait()

        def acc_body(j, carry, fields=fields, first=(w == 0)):
            js = pl.ds(j * LANES, LANES)
            acc = jnp.zeros((LANES,), jnp.float32) if first else part_v[js]
            for l in fields:
                sl = pl.ds(l * CH + j * LANES, LANES)
                acc = acc + jnp.where(seq_idx[sl] != 0, seq_val[sl], 0.0)
            part_v[js] = acc
            return carry

        lax.fori_loop(0, NJ, acc_body, 0)

    pltpu.sync_copy(part_v, part_hbm.at[pl.ds(base, CH)])


@functools.partial(
    pl.kernel,
    mesh=_mesh,
    out_type=jax.ShapeDtypeStruct((B,), jnp.float32),
    scratch_types=[
        pltpu.VMEM((_NTH * CH,), jnp.int32),
        pltpu.VMEM((_NTH * CH,), jnp.float32),
        pltpu.VMEM((NF * CH,), jnp.float32),
        pltpu.VMEM((CH,), jnp.float32),
        pltpu.VMEM((NF * LANES,), jnp.float32),
        pltpu.VMEM((LANES,), jnp.float32),
    ] + [pltpu.SemaphoreType.DMA] * len(_WAVES_T),
)
def _fm_tok_a(tf_hbm, *rest):
    tabs = rest[:_NTH]
    (ff_hbm, fw_hbm, bias_hbm, out_hbm,
     tok_idx, tok_val, ff_v, out_v, fw_v, bias_v) = rest[_NTH:_NTH + 10]
    sems = rest[_NTH + 10:]
    wid = lax.axis_index("s") * NC + lax.axis_index("c")
    base = wid * CH

    pltpu.sync_copy(tf_hbm.at[wid, pl.ds(0, _NTH * CH)], tok_idx)
    waves = []
    for w, fields in enumerate(_WAVES_T):
        waves.append([
            pltpu.async_copy(tabs[t].at[tok_idx.at[pl.ds(t * CH, CH)]],
                             tok_val.at[pl.ds(t * CH, CH)], sems[w])
            for t in fields])

    pltpu.sync_copy(ff_hbm.at[wid], ff_v)
    pltpu.sync_copy(fw_hbm, fw_v)
    pltpu.sync_copy(bias_hbm, bias_v)

    def base_body(j, carry):
        js = pl.ds(j * LANES, LANES)
        acc = bias_v[pl.ds(0, LANES)]
        for f in range(NF):
            acc = acc + ff_v[pl.ds(f * CH + j * LANES, LANES)] * fw_v[pl.ds(f * LANES, LANES)]
        out_v[js] = acc
        return carry

    lax.fori_loop(0, NJ, base_body, 0)

    for w, fields in enumerate(_WAVES_T):
        for cp in waves[w]:
            cp.wait()

        def acc_body(j, carry, fields=fields):
            js = pl.ds(j * LANES, LANES)
            acc = out_v[js]
            for t in fields:
                acc = acc + tok_val[pl.ds(t * CH + j * LANES, LANES)]
            out_v[js] = acc
            return carry

        lax.fori_loop(0, NJ, acc_body, 0)

    pltpu.sync_copy(out_v, out_hbm.at[pl.ds(base, CH)])


@functools.partial(
    pl.kernel,
    mesh=_mesh,
    out_type=jax.ShapeDtypeStruct((B,), jnp.float32),
    scratch_types=[
        pltpu.VMEM((_NTH * CH,), jnp.int32),
        pltpu.VMEM((_NTH * CH,), jnp.float32),
        pltpu.VMEM((CH,), jnp.float32),
        pltpu.VMEM((CH,), jnp.float32),
    ] + [pltpu.SemaphoreType.DMA] * len(_WAVES_T),
)
def _fm_tok_b(tf_hbm, *rest):
    tabs = rest[:_NTH]
    (parta_hbm, partb_hbm, out_hbm,
     tok_idx, tok_val, part_v, out_v) = rest[_NTH:_NTH + 7]
    sems = rest[_NTH + 7:]
    wid = lax.axis_index("s") * NC + lax.axis_index("c")
    base = wid * CH

    pltpu.sync_copy(tf_hbm.at[wid, pl.ds(_NTH * CH, _NTH * CH)], tok_idx)
    waves = []
    for w, fields in enumerate(_WAVES_T):
        waves.append([
            pltpu.async_copy(tabs[t].at[tok_idx.at[pl.ds(t * CH, CH)]],
                             tok_val.at[pl.ds(t * CH, CH)], sems[w])
            for t in fields])

    pltpu.sync_copy(parta_hbm.at[pl.ds(base, CH)], part_v)
    pltpu.sync_copy(partb_hbm.at[pl.ds(base, CH)], out_v)

    def base_body(j, carry):
        js = pl.ds(j * LANES, LANES)
        out_v[js] = out_v[js] + part_v[js]
        return carry

    lax.fori_loop(0, NJ, base_body, 0)

    for w, fields in enumerate(_WAVES_T):
        for cp in waves[w]:
            cp.wait()

        def acc_body(j, carry, fields=fields):
            js = pl.ds(j * LANES, LANES)
            acc = out_v[js]
            for t in fields:
                acc = acc + tok_val[pl.ds(t * CH + j * LANES, LANES)]
            out_v[js] = acc
            return carry

        lax.fori_loop(0, NJ, acc_body, 0)

    pltpu.sync_copy(out_v, out_hbm.at[pl.ds(base, CH)])


def _field_major(x, nfields):
    return x.T.reshape(nfields, NW, CH).transpose(1, 0, 2).reshape(NW, nfields * CH)


def kernel(float_fields, token_fields, token_seq_field, float_emb_table,
           token_emb_table, token_seq_emb_table, bias, offsets):
    tf_w = _field_major(token_fields, NT)
    sf_w = _field_major(token_seq_field, LS)
    ff_w = _field_major(float_fields, NF)
    fw_rep = jnp.repeat(float_emb_table.reshape(-1), LANES)
    bias_rep = jnp.broadcast_to(bias.reshape(1), (LANES,))
    del offsets
    part = _fm_seq(sf_w, token_seq_emb_table.reshape(-1))
    tabs = [token_emb_table[i * VT:(i + 1) * VT].reshape(-1)
            for i in range(NT)]
    part_a = _fm_tok_a(tf_w, *tabs[:_NTH], ff_w, fw_rep, bias_rep)
    out = _fm_tok_b(tf_w, *tabs[_NTH:], part, part_a)
    return out.reshape(B, 1)
